# baseline (speedup 1.0000x reference)
import functools

import jax
import jax.numpy as jnp
from jax import lax
from jax.experimental import pallas as pl
from jax.experimental.pallas import tpu as pltpu
from jax.experimental.pallas import tpu_sc as plsc

IN_F = 4096
OUT_F = 4096
NEG_SLOPE = 0.01

SFLAT = IN_F * OUT_F
NCHUNK = 16
CHUNK = SFLAT // NCHUNK
TRASH = CHUNK
NSUB = 16
NCORE = 2
SHARD_VECS = 656
SHARD = SHARD_VECS * 16
NNZ_PAD = SHARD * NSUB
SLICE = CHUNK // NSUB
ZBUF = 16384
BLK = 128


def _sc_densify_body(rows_hbm, cols_hbm, vals_hbm, s_hbm,
                     rows_v, cols_v, vals_v, sidx_v, zbuf, chunk, sem):
    c = lax.axis_index("c")
    s = lax.axis_index("s")
    shard0 = s * SHARD

    pltpu.async_copy(rows_hbm.at[pl.ds(shard0, SHARD)], rows_v, sem).wait()
    pltpu.async_copy(cols_hbm.at[pl.ds(shard0, SHARD)], cols_v, sem).wait()
    pltpu.async_copy(vals_hbm.at[pl.ds(shard0, SHARD)], vals_v, sem).wait()

    def zb(i, _):
        zbuf[pl.ds(i * 16, 16)] = jnp.zeros((16,), jnp.float32)
        return 0
    lax.fori_loop(0, ZBUF // 16, zb, 0)

    for q in range(NCHUNK // NCORE):
        chunk_id = q * NCORE + c
        base = chunk_id * CHUNK

        for z in range(SLICE // ZBUF):
            pltpu.sync_copy(zbuf, chunk.at[pl.ds(s * SLICE + z * ZBUF, ZBUF)])
        plsc.subcore_barrier()

        def scan(i, _):
            r = rows_v[pl.ds(i * 16, 16)]
            cc = cols_v[pl.ds(i * 16, 16)]
            rel = cc * OUT_F + r - base
            m = (rel >= 0) & (rel < CHUNK)
            sidx_v[pl.ds(i * 16, 16)] = jnp.where(m, rel, TRASH)
            return 0
        lax.fori_loop(0, SHARD_VECS, scan, 0)

        def scat(b, _):
            pltpu.sync_copy(vals_v.at[pl.ds(b * BLK, BLK)],
                            chunk.at[sidx_v.at[pl.ds(b * BLK, BLK)]],
                            add=True)
            return 0
        lax.fori_loop(0, SHARD // BLK, scat, 0)
        plsc.subcore_barrier()

        pltpu.sync_copy(chunk.at[pl.ds(s * SLICE, SLICE)],
                        s_hbm.at[pl.ds(base + s * SLICE, SLICE)])


_sc_densify = functools.partial(
    pl.kernel,
    out_type=jax.ShapeDtypeStruct((SFLAT,), jnp.float32),
    mesh=plsc.VectorSubcoreMesh(core_axis_name="c", subcore_axis_name="s"),
    scratch_types=[
        pltpu.VMEM((SHARD,), jnp.int32),
        pltpu.VMEM((SHARD,), jnp.int32),
        pltpu.VMEM((SHARD,), jnp.float32),
        pltpu.VMEM((SHARD,), jnp.int32),
        pltpu.VMEM((ZBUF,), jnp.float32),
        pltpu.VMEM_SHARED((CHUNK + 8,), jnp.float32),
        pltpu.SemaphoreType.DMA,
    ],
)(_sc_densify_body)


BN = 512


def _mm_body(a_ref, b_ref, bias_ref, o_ref):
    acc = jnp.dot(a_ref[...], b_ref[...], preferred_element_type=jnp.float32)
    acc = acc + bias_ref[...]
    o_ref[...] = jnp.where(acc >= 0, acc, NEG_SLOPE * acc)


def _matmul(features, s, bias2d):
    batch = features.shape[0]
    return pl.pallas_call(
        _mm_body,
        grid=(OUT_F // BN,),
        in_specs=[
            pl.BlockSpec((batch, IN_F), lambda j: (0, 0)),
            pl.BlockSpec((IN_F, BN), lambda j: (0, j)),
            pl.BlockSpec((1, BN), lambda j: (0, j)),
        ],
        out_specs=pl.BlockSpec((batch, BN), lambda j: (0, j)),
        out_shape=jax.ShapeDtypeStruct((batch, OUT_F), jnp.float32),
    )(features, s, bias2d)


def kernel(features, values, bias, rows, cols):
    nnz = rows.shape[0]
    pad = NNZ_PAD - nnz
    rows_p = jnp.concatenate([rows.astype(jnp.int32),
                              jnp.zeros((pad,), jnp.int32)])
    cols_p = jnp.concatenate([cols.astype(jnp.int32),
                              jnp.full((pad,), IN_F, jnp.int32)])
    vals_p = jnp.concatenate([values, jnp.zeros((pad,), jnp.float32)])
    s_flat = _sc_densify(rows_p, cols_p, vals_p)
    s = s_flat.reshape(IN_F, OUT_F)
    return _matmul(features, s, bias.reshape(1, OUT_F))


# speedup vs baseline: 1.0141x; 1.0141x over previous
import functools

import jax
import jax.numpy as jnp
from jax import lax
from jax.experimental import pallas as pl
from jax.experimental.pallas import tpu as pltpu
from jax.experimental.pallas import tpu_sc as plsc

IN_F = 4096
OUT_F = 4096
NEG_SLOPE = 0.01

SFLAT = IN_F * OUT_F
NCHUNK = 16
CHUNK = SFLAT // NCHUNK
TRASH = CHUNK
NSUB = 16
NCORE = 2
SHARD_VECS = 656
SHARD = SHARD_VECS * 16
NNZ_PAD = SHARD * NSUB
SLICE = CHUNK // NSUB
ZBUF = 16384
BLK = 128
NBLK = SHARD // BLK


def _sc_densify_body(rows_hbm, cols_hbm, vals_hbm, s_hbm,
                     rows_v, cols_v, vals_v, sidx_v, zbuf, chunk, sem, zsem, osem):
    c = lax.axis_index("c")
    s = lax.axis_index("s")
    shard0 = s * SHARD

    pltpu.async_copy(rows_hbm.at[pl.ds(shard0, SHARD)], rows_v, sem)
    pltpu.async_copy(cols_hbm.at[pl.ds(shard0, SHARD)], cols_v, sem)
    pltpu.async_copy(vals_hbm.at[pl.ds(shard0, SHARD)], vals_v, sem)

    def zb(i, _):
        zbuf[pl.ds(i * 16, 16)] = jnp.zeros((16,), jnp.float32)
        return 0
    lax.fori_loop(0, ZBUF // 16, zb, 0)

    pltpu.make_async_copy(rows_hbm.at[pl.ds(shard0, SHARD)], rows_v, sem).wait()
    pltpu.make_async_copy(cols_hbm.at[pl.ds(shard0, SHARD)], cols_v, sem).wait()
    pltpu.make_async_copy(vals_hbm.at[pl.ds(shard0, SHARD)], vals_v, sem).wait()

    for q in range(NCHUNK // NCORE):
        chunk_id = q * NCORE + c
        base = chunk_id * CHUNK

        if q > 0:
            prev = (q - 1) * NCORE + c
            pltpu.make_async_copy(
                chunk.at[pl.ds(s * SLICE, SLICE)],
                s_hbm.at[pl.ds(prev * CHUNK + s * SLICE, SLICE)], osem).wait()

        for z in range(SLICE // ZBUF):
            pltpu.async_copy(zbuf, chunk.at[pl.ds(s * SLICE + z * ZBUF, ZBUF)], zsem)

        def scan(i, _):
            r = rows_v[pl.ds(i * 16, 16)]
            cc = cols_v[pl.ds(i * 16, 16)]
            rel = cc * OUT_F + r - base
            m = (rel >= 0) & (rel < CHUNK)
            sidx_v[pl.ds(i * 16, 16)] = jnp.where(m, rel, TRASH)
            return 0
        lax.fori_loop(0, SHARD_VECS, scan, 0)

        for z in range(SLICE // ZBUF):
            pltpu.make_async_copy(
                zbuf, chunk.at[pl.ds(s * SLICE + z * ZBUF, ZBUF)], zsem).wait()
        plsc.subcore_barrier()

        def scat(b, _):
            pltpu.async_copy(vals_v.at[pl.ds(b * BLK, BLK)],
                             chunk.at[sidx_v.at[pl.ds(b * BLK, BLK)]],
                             sem, add=True)
            return 0
        lax.fori_loop(0, NBLK, scat, 0)

        def scat_drain(b, _):
            pltpu.make_async_copy(vals_v.at[pl.ds(b * BLK, BLK)],
                                  chunk.at[sidx_v.at[pl.ds(b * BLK, BLK)]],
                                  sem).wait()
            return 0
        lax.fori_loop(0, NBLK, scat_drain, 0)
        plsc.subcore_barrier()

        pltpu.async_copy(chunk.at[pl.ds(s * SLICE, SLICE)],
                         s_hbm.at[pl.ds(base + s * SLICE, SLICE)], osem)

    last = (NCHUNK // NCORE - 1) * NCORE + c
    pltpu.make_async_copy(chunk.at[pl.ds(s * SLICE, SLICE)],
                          s_hbm.at[pl.ds(last * CHUNK + s * SLICE, SLICE)],
                          osem).wait()


_sc_densify = functools.partial(
    pl.kernel,
    out_type=jax.ShapeDtypeStruct((SFLAT,), jnp.float32),
    mesh=plsc.VectorSubcoreMesh(core_axis_name="c", subcore_axis_name="s"),
    scratch_types=[
        pltpu.VMEM((SHARD,), jnp.int32),
        pltpu.VMEM((SHARD,), jnp.int32),
        pltpu.VMEM((SHARD,), jnp.float32),
        pltpu.VMEM((SHARD,), jnp.int32),
        pltpu.VMEM((ZBUF,), jnp.float32),
        pltpu.VMEM_SHARED((CHUNK + 8,), jnp.float32),
        pltpu.SemaphoreType.DMA,
        pltpu.SemaphoreType.DMA,
        pltpu.SemaphoreType.DMA,
    ],
)(_sc_densify_body)


BN = 512


def _mm_body(a_ref, b_ref, bias_ref, o_ref):
    acc = jnp.dot(a_ref[...], b_ref[...], preferred_element_type=jnp.float32)
    acc = acc + bias_ref[...]
    o_ref[...] = jnp.where(acc >= 0, acc, NEG_SLOPE * acc)


def _matmul(features, s, bias2d):
    batch = features.shape[0]
    return pl.pallas_call(
        _mm_body,
        grid=(OUT_F // BN,),
        in_specs=[
            pl.BlockSpec((batch, IN_F), lambda j: (0, 0)),
            pl.BlockSpec((IN_F, BN), lambda j: (0, j)),
            pl.BlockSpec((1, BN), lambda j: (0, j)),
        ],
        out_specs=pl.BlockSpec((batch, BN), lambda j: (0, j)),
        out_shape=jax.ShapeDtypeStruct((batch, OUT_F), jnp.float32),
    )(features, s, bias2d)


def kernel(features, values, bias, rows, cols):
    nnz = rows.shape[0]
    pad = NNZ_PAD - nnz
    rows_p = jnp.concatenate([rows.astype(jnp.int32),
                              jnp.zeros((pad,), jnp.int32)])
    cols_p = jnp.concatenate([cols.astype(jnp.int32),
                              jnp.full((pad,), IN_F, jnp.int32)])
    vals_p = jnp.concatenate([values, jnp.zeros((pad,), jnp.float32)])
    s_flat = _sc_densify(rows_p, cols_p, vals_p)
    s = s_flat.reshape(IN_F, OUT_F)
    return _matmul(features, s, bias.reshape(1, OUT_F))


# speedup vs baseline: 6.2353x; 6.1487x over previous
import functools

import jax
import jax.numpy as jnp
from jax import lax
from jax.experimental import pallas as pl
from jax.experimental.pallas import tpu as pltpu
from jax.experimental.pallas import tpu_sc as plsc

IN_F = 4096
OUT_F = 4096
NEG_SLOPE = 0.01

SFLAT = IN_F * OUT_F
NCHUNK = 16
CHUNK = SFLAT // NCHUNK
NPASS = NCHUNK // 2
NSUB = 16
NCORE = 2
SHARD_VECS = 656
SHARD = SHARD_VECS * 16
NNZ_PAD = SHARD * NSUB
SLICE = CHUNK // NSUB
ZBUF = 4096
BLK = 128
BINCAP = SHARD + NPASS * (BLK - 1) + 112
BINCAP = ((BINCAP + BLK - 1) // BLK) * BLK
SPTILE = BINCAP + BLK
WCAP = BINCAP + BLK
BIGI = max(2 * SHARD, WCAP)
BIGF = max(SHARD, WCAP)


def _sc_densify_body(rows_hbm, cols_hbm, vals_hbm, s_hbm,
                     big_i, big_f, zbuf, tmp, offs,
                     sp_bidx, sp_bval, chunk, sem, zsem, osem):
    c = lax.axis_index("c")
    s = lax.axis_index("s")
    shard0 = s * SHARD
    lane = lax.iota(jnp.int32, 16)
    rbase = s * SPTILE

    pltpu.async_copy(rows_hbm.at[pl.ds(shard0, SHARD)],
                     big_i.at[pl.ds(0, SHARD)], sem)
    pltpu.async_copy(cols_hbm.at[pl.ds(shard0, SHARD)],
                     big_i.at[pl.ds(SHARD, SHARD)], sem)
    pltpu.async_copy(vals_hbm.at[pl.ds(shard0, SHARD)],
                     big_f.at[pl.ds(0, SHARD)], sem)

    def zb(i, _):
        zbuf[pl.ds(i * 16, 16)] = jnp.zeros((16,), jnp.float32)
        return 0
    lax.fori_loop(0, ZBUF // 16, zb, 0)

    pltpu.make_async_copy(rows_hbm.at[pl.ds(shard0, SHARD)],
                          big_i.at[pl.ds(0, SHARD)], sem).wait()
    pltpu.make_async_copy(cols_hbm.at[pl.ds(shard0, SHARD)],
                          big_i.at[pl.ds(SHARD, SHARD)], sem).wait()
    pltpu.make_async_copy(vals_hbm.at[pl.ds(shard0, SHARD)],
                          big_f.at[pl.ds(0, SHARD)], sem).wait()

    def ha(i, cnts):
        r = big_i[pl.ds(i * 16, 16)]
        cc = big_i[pl.ds(SHARD + i * 16, 16)]
        ch = (cc * OUT_F + r) >> 20
        one = jnp.ones((16,), jnp.int32)
        zero = jnp.zeros((16,), jnp.int32)
        return tuple(cnts[q] + jnp.where(ch == (q * NCORE + c), one, zero)
                     for q in range(NPASS))
    cnts = lax.fori_loop(0, SHARD_VECS, ha,
                         tuple(jnp.zeros((16,), jnp.int32)
                               for _ in range(NPASS)))

    starts = []
    bq = jnp.int32(0)
    for q in range(NPASS):
        x = cnts[q]
        for sh in (1, 2, 4, 8):
            tmp[pl.ds(16, 16)] = x
            tmp[pl.ds(0, 16)] = jnp.zeros((16,), jnp.int32)
            x = x + tmp[pl.ds(16 - sh, 16)]
        tmp[pl.ds(16, 16)] = x
        tmp[pl.ds(0, 16)] = jnp.zeros((16,), jnp.int32)
        excl = tmp[pl.ds(15, 16)]
        tot = x[15]
        offs[q] = bq
        offs[NPASS + 1 + q] = tot
        starts.append(excl + (rbase + bq * BLK))
        bq = bq + ((tot + BLK - 1) >> 7)
    offs[NPASS] = bq

    dump = rbase + BINCAP + lane

    def hb(i, sts):
        r = big_i[pl.ds(i * 16, 16)]
        cc = big_i[pl.ds(SHARD + i * 16, 16)]
        flat = cc * OUT_F + r
        ch = flat >> 20
        rel = flat & (CHUNK - 1)
        one = jnp.ones((16,), jnp.int32)
        zero = jnp.zeros((16,), jnp.int32)
        pos = dump
        out = []
        for q in range(NPASS):
            m = ch == (q * NCORE + c)
            pos = jnp.where(m, sts[q], pos)
            out.append(sts[q] + jnp.where(m, one, zero))
        big_i[pl.ds(i * 16, 16)] = rel
        big_i[pl.ds(SHARD + i * 16, 16)] = pos
        return tuple(out)
    lax.fori_loop(0, SHARD_VECS, hb, tuple(starts))

    def pfire(b, _):
        idx = big_i.at[pl.ds(SHARD + b * BLK, BLK)]
        pltpu.async_copy(big_i.at[pl.ds(b * BLK, BLK)], sp_bidx.at[idx], sem)
        pltpu.async_copy(big_f.at[pl.ds(b * BLK, BLK)], sp_bval.at[idx], sem)
        return 0
    lax.fori_loop(0, SHARD // BLK, pfire, 0)

    def pdrain(b, _):
        idx = big_i.at[pl.ds(SHARD + b * BLK, BLK)]
        pltpu.make_async_copy(big_i.at[pl.ds(b * BLK, BLK)],
                              sp_bidx.at[idx], sem).wait()
        pltpu.make_async_copy(big_f.at[pl.ds(b * BLK, BLK)],
                              sp_bval.at[idx], sem).wait()
        return 0
    lax.fori_loop(0, SHARD // BLK, pdrain, 0)

    def _settle(i, acc):
        return acc + i * i
    offs[2 * NPASS + 1] = lax.fori_loop(0, 1000, _settle, jnp.int32(0))
    plsc.subcore_barrier()

    def chunk_pass(q, _):
        chunk_id = q * NCORE + c
        base = chunk_id * CHUNK

        @pl.when(q > 0)
        def _():
            prev = (q - 1) * NCORE + c
            pltpu.make_async_copy(
                chunk.at[pl.ds(s * SLICE, SLICE)],
                s_hbm.at[pl.ds(prev * CHUNK + s * SLICE, SLICE)], osem).wait()

        for z in range(SLICE // ZBUF):
            pltpu.async_copy(zbuf, chunk.at[pl.ds(s * SLICE + z * ZBUF, ZBUF)],
                             zsem)
        @pl.when(s == 0)
        def _():
            pltpu.async_copy(zbuf.at[pl.ds(0, BLK)],
                             chunk.at[pl.ds(CHUNK, BLK)], zsem)

        b_lo = offs[q]
        tot = offs[NPASS + 1 + q]
        nblk = (tot + BLK - 1) >> 7

        def cin(b, _):
            pltpu.async_copy(
                sp_bidx.at[pl.ds(rbase + (b_lo + b) * BLK, BLK)],
                big_i.at[pl.ds(b * BLK, BLK)], sem)
            pltpu.async_copy(
                sp_bval.at[pl.ds(rbase + (b_lo + b) * BLK, BLK)],
                big_f.at[pl.ds(b * BLK, BLK)], sem)
            return 0
        lax.fori_loop(0, nblk, cin, 0)

        def cin_drain(b, _):
            pltpu.make_async_copy(
                sp_bidx.at[pl.ds(rbase + (b_lo + b) * BLK, BLK)],
                big_i.at[pl.ds(b * BLK, BLK)], sem).wait()
            pltpu.make_async_copy(
                sp_bval.at[pl.ds(rbase + (b_lo + b) * BLK, BLK)],
                big_f.at[pl.ds(b * BLK, BLK)], sem).wait()
            return 0
        lax.fori_loop(0, nblk, cin_drain, 0)

        def tfill(j, _):
            p = tot + j * 16
            big_i[pl.ds(p, 16)] = CHUNK + ((p + lane) & (BLK - 1))
            return 0
        lax.fori_loop(0, BLK // 16, tfill, 0)

        for z in range(SLICE // ZBUF):
            pltpu.make_async_copy(
                zbuf, chunk.at[pl.ds(s * SLICE + z * ZBUF, ZBUF)], zsem).wait()

        @pl.when(s == 0)
        def _():
            pltpu.make_async_copy(zbuf.at[pl.ds(0, BLK)],
                                  chunk.at[pl.ds(CHUNK, BLK)], zsem).wait()
        plsc.subcore_barrier()

        for w in range(NSUB):
            @pl.when(s == w)
            def _():
                def fire(b, _):
                    pltpu.async_copy(
                        big_f.at[pl.ds(b * BLK, BLK)],
                        chunk.at[big_i.at[pl.ds(b * BLK, BLK)]],
                        sem, add=True)
                    return 0
                lax.fori_loop(0, nblk, fire, 0)

                def drain(b, _):
                    pltpu.make_async_copy(
                        big_f.at[pl.ds(b * BLK, BLK)],
                        chunk.at[big_i.at[pl.ds(b * BLK, BLK)]],
                        sem).wait()
                    return 0
                lax.fori_loop(0, nblk, drain, 0)
            plsc.subcore_barrier()

        pltpu.async_copy(chunk.at[pl.ds(s * SLICE, SLICE)],
                         s_hbm.at[pl.ds(base + s * SLICE, SLICE)], osem)
        return 0

    lax.fori_loop(0, NPASS, chunk_pass, 0)

    last = (NPASS - 1) * NCORE + c
    pltpu.make_async_copy(chunk.at[pl.ds(s * SLICE, SLICE)],
                          s_hbm.at[pl.ds(last * CHUNK + s * SLICE, SLICE)],
                          osem).wait()


_sc_densify = functools.partial(
    pl.kernel,
    out_type=jax.ShapeDtypeStruct((SFLAT,), jnp.float32),
    mesh=plsc.VectorSubcoreMesh(core_axis_name="c", subcore_axis_name="s"),
    scratch_types=[
        pltpu.VMEM((BIGI,), jnp.int32),
        pltpu.VMEM((BIGF,), jnp.float32),
        pltpu.VMEM((ZBUF,), jnp.float32),
        pltpu.VMEM((48,), jnp.int32),
        pltpu.SMEM((2 * NPASS + 4,), jnp.int32),
        pltpu.VMEM_SHARED((NSUB * SPTILE,), jnp.int32),
        pltpu.VMEM_SHARED((NSUB * SPTILE,), jnp.float32),
        pltpu.VMEM_SHARED((CHUNK + BLK,), jnp.float32),
        pltpu.SemaphoreType.DMA,
        pltpu.SemaphoreType.DMA,
        pltpu.SemaphoreType.DMA,
    ],
)(_sc_densify_body)


BN = 512


def _mm_body(a_ref, b_ref, bias_ref, o_ref):
    acc = jnp.dot(a_ref[...], b_ref[...], preferred_element_type=jnp.float32)
    acc = acc + bias_ref[...]
    o_ref[...] = jnp.where(acc >= 0, acc, NEG_SLOPE * acc)


def _matmul(features, s, bias2d):
    batch = features.shape[0]
    return pl.pallas_call(
        _mm_body,
        grid=(OUT_F // BN,),
        in_specs=[
            pl.BlockSpec((batch, IN_F), lambda j: (0, 0)),
            pl.BlockSpec((IN_F, BN), lambda j: (0, j)),
            pl.BlockSpec((1, BN), lambda j: (0, j)),
        ],
        out_specs=pl.BlockSpec((batch, BN), lambda j: (0, j)),
        out_shape=jax.ShapeDtypeStruct((batch, OUT_F), jnp.float32),
    )(features, s, bias2d)


def kernel(features, values, bias, rows, cols):
    nnz = rows.shape[0]
    pad = NNZ_PAD - nnz
    rows_p = jnp.concatenate([rows.astype(jnp.int32),
                              jnp.zeros((pad,), jnp.int32)])
    cols_p = jnp.concatenate([cols.astype(jnp.int32),
                              jnp.full((pad,), IN_F, jnp.int32)])
    vals_p = jnp.concatenate([values, jnp.zeros((pad,), jnp.float32)])
    s_flat = _sc_densify(rows_p, cols_p, vals_p)
    s = s_flat.reshape(IN_F, OUT_F)
    return _matmul(features, s, bias.reshape(1, OUT_F))


# speedup vs baseline: 6.2505x; 1.0024x over previous
import functools

import jax
import jax.numpy as jnp
from jax import lax
from jax.experimental import pallas as pl
from jax.experimental.pallas import tpu as pltpu
from jax.experimental.pallas import tpu_sc as plsc

IN_F = 4096
OUT_F = 4096
NEG_SLOPE = 0.01

SFLAT = IN_F * OUT_F
NCHUNK = 16
CHUNK = SFLAT // NCHUNK
NPASS = NCHUNK // 2
NSUB = 16
NCORE = 2
SHARD_VECS = 656
SHARD = SHARD_VECS * 16
NNZ_PAD = SHARD * NSUB
SLICE = CHUNK // NSUB
ZBUF = 4096
BLK = 128
BINCAP = SHARD + NPASS * (BLK - 1) + 112
BINCAP = ((BINCAP + BLK - 1) // BLK) * BLK
SPTILE = BINCAP + BLK
WCAP = BINCAP + BLK
BIGI = max(2 * SHARD, WCAP)
BIGF = max(SHARD, WCAP)


def _sc_densify_body(rows_hbm, cols_hbm, vals_hbm, s_hbm,
                     big_i, big_f, zbuf, tmp, offs,
                     sp_bidx, sp_bval, chunk, sem, zsem, osem):
    c = lax.axis_index("c")
    s = lax.axis_index("s")
    shard0 = s * SHARD
    lane = lax.iota(jnp.int32, 16)
    rbase = s * SPTILE

    pltpu.async_copy(rows_hbm.at[pl.ds(shard0, SHARD)],
                     big_i.at[pl.ds(0, SHARD)], sem)
    pltpu.async_copy(cols_hbm.at[pl.ds(shard0, SHARD)],
                     big_i.at[pl.ds(SHARD, SHARD)], sem)
    pltpu.async_copy(vals_hbm.at[pl.ds(shard0, SHARD)],
                     big_f.at[pl.ds(0, SHARD)], sem)

    def zb(i, _):
        zbuf[pl.ds(i * 16, 16)] = jnp.zeros((16,), jnp.float32)
        return 0
    lax.fori_loop(0, ZBUF // 16, zb, 0)

    pltpu.make_async_copy(rows_hbm.at[pl.ds(shard0, SHARD)],
                          big_i.at[pl.ds(0, SHARD)], sem).wait()
    pltpu.make_async_copy(cols_hbm.at[pl.ds(shard0, SHARD)],
                          big_i.at[pl.ds(SHARD, SHARD)], sem).wait()
    pltpu.make_async_copy(vals_hbm.at[pl.ds(shard0, SHARD)],
                          big_f.at[pl.ds(0, SHARD)], sem).wait()

    def ha(i, cnts):
        r = big_i[pl.ds(i * 16, 16)]
        cc = big_i[pl.ds(SHARD + i * 16, 16)]
        ch = (cc * OUT_F + r) >> 20
        one = jnp.ones((16,), jnp.int32)
        zero = jnp.zeros((16,), jnp.int32)
        return tuple(cnts[q] + jnp.where(ch == (q * NCORE + c), one, zero)
                     for q in range(NPASS))
    cnts = lax.fori_loop(0, SHARD_VECS, ha,
                         tuple(jnp.zeros((16,), jnp.int32)
                               for _ in range(NPASS)))

    starts = []
    bq = jnp.int32(0)
    for q in range(NPASS):
        x = cnts[q]
        for sh in (1, 2, 4, 8):
            tmp[pl.ds(16, 16)] = x
            tmp[pl.ds(0, 16)] = jnp.zeros((16,), jnp.int32)
            x = x + tmp[pl.ds(16 - sh, 16)]
        tmp[pl.ds(16, 16)] = x
        tmp[pl.ds(0, 16)] = jnp.zeros((16,), jnp.int32)
        excl = tmp[pl.ds(15, 16)]
        tot = x[15]
        offs[q] = bq
        offs[NPASS + 1 + q] = tot
        starts.append(excl + (rbase + bq * BLK))
        bq = bq + ((tot + BLK - 1) >> 7)
    offs[NPASS] = bq

    dump = rbase + BINCAP + lane

    def hb(i, sts):
        r = big_i[pl.ds(i * 16, 16)]
        cc = big_i[pl.ds(SHARD + i * 16, 16)]
        flat = cc * OUT_F + r
        ch = flat >> 20
        rel = flat & (CHUNK - 1)
        one = jnp.ones((16,), jnp.int32)
        zero = jnp.zeros((16,), jnp.int32)
        pos = dump
        out = []
        for q in range(NPASS):
            m = ch == (q * NCORE + c)
            pos = jnp.where(m, sts[q], pos)
            out.append(sts[q] + jnp.where(m, one, zero))
        big_i[pl.ds(i * 16, 16)] = rel
        big_i[pl.ds(SHARD + i * 16, 16)] = pos
        return tuple(out)
    lax.fori_loop(0, SHARD_VECS, hb, tuple(starts))

    def pfire(b, _):
        idx = big_i.at[pl.ds(SHARD + b * BLK, BLK)]
        pltpu.async_copy(big_i.at[pl.ds(b * BLK, BLK)], sp_bidx.at[idx], sem)
        pltpu.async_copy(big_f.at[pl.ds(b * BLK, BLK)], sp_bval.at[idx], sem)
        return 0
    lax.fori_loop(0, SHARD // BLK, pfire, 0)

    def pdrain(b, _):
        idx = big_i.at[pl.ds(SHARD + b * BLK, BLK)]
        pltpu.make_async_copy(big_i.at[pl.ds(b * BLK, BLK)],
                              sp_bidx.at[idx], sem).wait()
        pltpu.make_async_copy(big_f.at[pl.ds(b * BLK, BLK)],
                              sp_bval.at[idx], sem).wait()
        return 0
    lax.fori_loop(0, SHARD // BLK, pdrain, 0)

    def _settle(i, acc):
        return acc + i * i
    offs[2 * NPASS + 1] = lax.fori_loop(0, 1000, _settle, jnp.int32(0))
    plsc.subcore_barrier()

    def chunk_pass(q, _):
        chunk_id = q * NCORE + c
        base = chunk_id * CHUNK

        @pl.when(q > 0)
        def _():
            prev = (q - 1) * NCORE + c
            pltpu.make_async_copy(
                chunk.at[pl.ds(s * SLICE, SLICE)],
                s_hbm.at[pl.ds(prev * CHUNK + s * SLICE, SLICE)], osem).wait()

        for z in range(SLICE // ZBUF):
            pltpu.async_copy(zbuf, chunk.at[pl.ds(s * SLICE + z * ZBUF, ZBUF)],
                             zsem)
        @pl.when(s == 0)
        def _():
            pltpu.async_copy(zbuf.at[pl.ds(0, BLK)],
                             chunk.at[pl.ds(CHUNK, BLK)], zsem)

        b_lo = offs[q]
        tot = offs[NPASS + 1 + q]
        nblk = (tot + BLK - 1) >> 7

        def cin(b, _):
            pltpu.async_copy(
                sp_bidx.at[pl.ds(rbase + (b_lo + b) * BLK, BLK)],
                big_i.at[pl.ds(b * BLK, BLK)], sem)
            pltpu.async_copy(
                sp_bval.at[pl.ds(rbase + (b_lo + b) * BLK, BLK)],
                big_f.at[pl.ds(b * BLK, BLK)], sem)
            return 0
        lax.fori_loop(0, nblk, cin, 0)

        def cin_drain(b, _):
            pltpu.make_async_copy(
                sp_bidx.at[pl.ds(rbase + (b_lo + b) * BLK, BLK)],
                big_i.at[pl.ds(b * BLK, BLK)], sem).wait()
            pltpu.make_async_copy(
                sp_bval.at[pl.ds(rbase + (b_lo + b) * BLK, BLK)],
                big_f.at[pl.ds(b * BLK, BLK)], sem).wait()
            return 0
        lax.fori_loop(0, nblk, cin_drain, 0)

        def tfill(j, _):
            p = tot + j * 16
            big_i[pl.ds(p, 16)] = CHUNK + ((p + lane) & (BLK - 1))
            return 0
        lax.fori_loop(0, BLK // 16, tfill, 0)

        for z in range(SLICE // ZBUF):
            pltpu.make_async_copy(
                zbuf, chunk.at[pl.ds(s * SLICE + z * ZBUF, ZBUF)], zsem).wait()

        @pl.when(s == 0)
        def _():
            pltpu.make_async_copy(zbuf.at[pl.ds(0, BLK)],
                                  chunk.at[pl.ds(CHUNK, BLK)], zsem).wait()
        plsc.subcore_barrier()

        for w in range(NSUB):
            @pl.when(s == w)
            def _():
                def fire(b, _):
                    pltpu.async_copy(
                        big_f.at[pl.ds(b * BLK, BLK)],
                        chunk.at[big_i.at[pl.ds(b * BLK, BLK)]],
                        sem, add=True)
                    return 0
                lax.fori_loop(0, nblk, fire, 0)

                def drain(b, _):
                    pltpu.make_async_copy(
                        big_f.at[pl.ds(b * BLK, BLK)],
                        chunk.at[big_i.at[pl.ds(b * BLK, BLK)]],
                        sem).wait()
                    return 0
                lax.fori_loop(0, nblk, drain, 0)
            plsc.subcore_barrier()

        pltpu.async_copy(chunk.at[pl.ds(s * SLICE, SLICE)],
                         s_hbm.at[pl.ds(base + s * SLICE, SLICE)], osem)
        return 0

    lax.fori_loop(0, NPASS, chunk_pass, 0)

    last = (NPASS - 1) * NCORE + c
    pltpu.make_async_copy(chunk.at[pl.ds(s * SLICE, SLICE)],
                          s_hbm.at[pl.ds(last * CHUNK + s * SLICE, SLICE)],
                          osem).wait()


_sc_densify = functools.partial(
    pl.kernel,
    out_type=jax.ShapeDtypeStruct((SFLAT,), jnp.float32),
    mesh=plsc.VectorSubcoreMesh(core_axis_name="c", subcore_axis_name="s"),
    scratch_types=[
        pltpu.VMEM((BIGI,), jnp.int32),
        pltpu.VMEM((BIGF,), jnp.float32),
        pltpu.VMEM((ZBUF,), jnp.float32),
        pltpu.VMEM((48,), jnp.int32),
        pltpu.SMEM((2 * NPASS + 4,), jnp.int32),
        pltpu.VMEM_SHARED((NSUB * SPTILE,), jnp.int32),
        pltpu.VMEM_SHARED((NSUB * SPTILE,), jnp.float32),
        pltpu.VMEM_SHARED((CHUNK + BLK,), jnp.float32),
        pltpu.SemaphoreType.DMA,
        pltpu.SemaphoreType.DMA,
        pltpu.SemaphoreType.DMA,
    ],
)(_sc_densify_body)


BN = 512


def _mm_body(a_ref, b_ref, bias_ref, o_ref):
    a = a_ref[...].astype(jnp.bfloat16)
    b = b_ref[...].astype(jnp.bfloat16)
    acc = jnp.dot(a, b, preferred_element_type=jnp.float32)
    acc = acc + bias_ref[...]
    o_ref[...] = jnp.where(acc >= 0, acc, NEG_SLOPE * acc)


def _matmul(features, s, bias2d):
    batch = features.shape[0]
    return pl.pallas_call(
        _mm_body,
        grid=(OUT_F // BN,),
        in_specs=[
            pl.BlockSpec((batch, IN_F), lambda j: (0, 0)),
            pl.BlockSpec((IN_F, BN), lambda j: (0, j)),
            pl.BlockSpec((1, BN), lambda j: (0, j)),
        ],
        out_specs=pl.BlockSpec((batch, BN), lambda j: (0, j)),
        out_shape=jax.ShapeDtypeStruct((batch, OUT_F), jnp.float32),
    )(features, s, bias2d)


def kernel(features, values, bias, rows, cols):
    nnz = rows.shape[0]
    pad = NNZ_PAD - nnz
    rows_p = jnp.concatenate([rows.astype(jnp.int32),
                              jnp.zeros((pad,), jnp.int32)])
    cols_p = jnp.concatenate([cols.astype(jnp.int32),
                              jnp.full((pad,), IN_F, jnp.int32)])
    vals_p = jnp.concatenate([values, jnp.zeros((pad,), jnp.float32)])
    s_flat = _sc_densify(rows_p, cols_p, vals_p)
    s = s_flat.reshape(IN_F, OUT_F)
    return _matmul(features, s, bias.reshape(1, OUT_F))


# speedup vs baseline: 7.6835x; 1.2293x over previous
import functools

import jax
import jax.numpy as jnp
from jax import lax
from jax.experimental import pallas as pl
from jax.experimental.pallas import tpu as pltpu
from jax.experimental.pallas import tpu_sc as plsc

IN_F = 4096
OUT_F = 4096
NEG_SLOPE = 0.01

SFLAT = IN_F * OUT_F
NCHUNK = 16
CHUNK = SFLAT // NCHUNK
NPASS = NCHUNK // 2
NSUB = 16
NCORE = 2
SHARD_VECS = 656
SHARD = SHARD_VECS * 16
NNZ_PAD = SHARD * NSUB
SLICE = CHUNK // NSUB
ZBUF = 4096
BLK = 128
BINCAP = SHARD + NPASS * (BLK - 1) + 112
BINCAP = ((BINCAP + BLK - 1) // BLK) * BLK
SPTILE = BINCAP + BLK
WCAP = BINCAP + BLK
BIGI = max(2 * SHARD, WCAP)
BIGF = max(SHARD, WCAP)


def _sc_densify_body(rows_hbm, cols_hbm, vals_hbm, s_hbm,
                     big_i, big_f, zbuf, tmp, offs,
                     sp_bidx, sp_bval, chunk, sem, zsem, osem):
    c = lax.axis_index("c")
    s = lax.axis_index("s")
    shard0 = s * SHARD
    lane = lax.iota(jnp.int32, 16)
    rbase = s * SPTILE

    pltpu.async_copy(rows_hbm.at[pl.ds(shard0, SHARD)],
                     big_i.at[pl.ds(0, SHARD)], sem)
    pltpu.async_copy(cols_hbm.at[pl.ds(shard0, SHARD)],
                     big_i.at[pl.ds(SHARD, SHARD)], sem)
    pltpu.async_copy(vals_hbm.at[pl.ds(shard0, SHARD)],
                     big_f.at[pl.ds(0, SHARD)], sem)

    def zb(i, _):
        zbuf[pl.ds(i * 16, 16)] = jnp.zeros((16,), jnp.float32)
        return 0
    lax.fori_loop(0, ZBUF // 16, zb, 0)

    pltpu.make_async_copy(rows_hbm.at[pl.ds(shard0, SHARD)],
                          big_i.at[pl.ds(0, SHARD)], sem).wait()
    pltpu.make_async_copy(cols_hbm.at[pl.ds(shard0, SHARD)],
                          big_i.at[pl.ds(SHARD, SHARD)], sem).wait()
    pltpu.make_async_copy(vals_hbm.at[pl.ds(shard0, SHARD)],
                          big_f.at[pl.ds(0, SHARD)], sem).wait()

    def ha(i, cnts):
        r = big_i[pl.ds(i * 16, 16)]
        cc = big_i[pl.ds(SHARD + i * 16, 16)]
        ch = (cc * OUT_F + r) >> 20
        one = jnp.ones((16,), jnp.int32)
        zero = jnp.zeros((16,), jnp.int32)
        return tuple(cnts[q] + jnp.where(ch == (q * NCORE + c), one, zero)
                     for q in range(NPASS))
    cnts = lax.fori_loop(0, SHARD_VECS, ha,
                         tuple(jnp.zeros((16,), jnp.int32)
                               for _ in range(NPASS)))

    starts = []
    bq = jnp.int32(0)
    for q in range(NPASS):
        x = cnts[q]
        for sh in (1, 2, 4, 8):
            tmp[pl.ds(16, 16)] = x
            tmp[pl.ds(0, 16)] = jnp.zeros((16,), jnp.int32)
            x = x + tmp[pl.ds(16 - sh, 16)]
        tmp[pl.ds(16, 16)] = x
        tmp[pl.ds(0, 16)] = jnp.zeros((16,), jnp.int32)
        excl = tmp[pl.ds(15, 16)]
        tot = x[15]
        offs[q] = bq
        offs[NPASS + 1 + q] = tot
        starts.append(excl + (rbase + bq * BLK))
        bq = bq + ((tot + BLK - 1) >> 7)
    offs[NPASS] = bq

    dump = rbase + BINCAP + lane

    def hb(i, sts):
        r = big_i[pl.ds(i * 16, 16)]
        cc = big_i[pl.ds(SHARD + i * 16, 16)]
        flat = cc * OUT_F + r
        ch = flat >> 20
        rel = flat & (CHUNK - 1)
        one = jnp.ones((16,), jnp.int32)
        zero = jnp.zeros((16,), jnp.int32)
        pos = dump
        out = []
        for q in range(NPASS):
            m = ch == (q * NCORE + c)
            pos = jnp.where(m, sts[q], pos)
            out.append(sts[q] + jnp.where(m, one, zero))
        big_i[pl.ds(i * 16, 16)] = rel
        big_i[pl.ds(SHARD + i * 16, 16)] = pos
        return tuple(out)
    lax.fori_loop(0, SHARD_VECS, hb, tuple(starts))

    def pfire(b, _):
        idx = big_i.at[pl.ds(SHARD + b * BLK, BLK)]
        pltpu.async_copy(big_i.at[pl.ds(b * BLK, BLK)], sp_bidx.at[idx], sem)
        pltpu.async_copy(big_f.at[pl.ds(b * BLK, BLK)], sp_bval.at[idx], sem)
        return 0
    lax.fori_loop(0, SHARD // BLK, pfire, 0)

    def pdrain(b, _):
        idx = big_i.at[pl.ds(SHARD + b * BLK, BLK)]
        pltpu.make_async_copy(big_i.at[pl.ds(b * BLK, BLK)],
                              sp_bidx.at[idx], sem).wait()
        pltpu.make_async_copy(big_f.at[pl.ds(b * BLK, BLK)],
                              sp_bval.at[idx], sem).wait()
        return 0
    lax.fori_loop(0, SHARD // BLK, pdrain, 0)

    def _settle(i, acc):
        return acc + i * i
    offs[2 * NPASS + 1] = lax.fori_loop(0, 1000, _settle, jnp.int32(0))
    plsc.subcore_barrier()

    def chunk_pass(q, _):
        chunk_id = q * NCORE + c
        base = chunk_id * CHUNK

        @pl.when(q > 0)
        def _():
            prev = (q - 1) * NCORE + c
            pltpu.make_async_copy(
                chunk.at[pl.ds(s * SLICE, SLICE)],
                s_hbm.at[pl.ds(prev * CHUNK + s * SLICE, SLICE)], osem).wait()

        for z in range(SLICE // ZBUF):
            pltpu.async_copy(zbuf, chunk.at[pl.ds(s * SLICE + z * ZBUF, ZBUF)],
                             zsem)
        @pl.when(s == 0)
        def _():
            pltpu.async_copy(zbuf.at[pl.ds(0, BLK)],
                             chunk.at[pl.ds(CHUNK, BLK)], zsem)

        b_lo = offs[q]
        tot = offs[NPASS + 1 + q]
        nblk = (tot + BLK - 1) >> 7

        def cin(b, _):
            pltpu.async_copy(
                sp_bidx.at[pl.ds(rbase + (b_lo + b) * BLK, BLK)],
                big_i.at[pl.ds(b * BLK, BLK)], sem)
            pltpu.async_copy(
                sp_bval.at[pl.ds(rbase + (b_lo + b) * BLK, BLK)],
                big_f.at[pl.ds(b * BLK, BLK)], sem)
            return 0
        lax.fori_loop(0, nblk, cin, 0)

        def cin_drain(b, _):
            pltpu.make_async_copy(
                sp_bidx.at[pl.ds(rbase + (b_lo + b) * BLK, BLK)],
                big_i.at[pl.ds(b * BLK, BLK)], sem).wait()
            pltpu.make_async_copy(
                sp_bval.at[pl.ds(rbase + (b_lo + b) * BLK, BLK)],
                big_f.at[pl.ds(b * BLK, BLK)], sem).wait()
            return 0
        lax.fori_loop(0, nblk, cin_drain, 0)

        def tfill(j, _):
            p = tot + j * 16
            big_i[pl.ds(p, 16)] = CHUNK + ((p + lane) & (BLK - 1))
            return 0
        lax.fori_loop(0, BLK // 16, tfill, 0)

        for z in range(SLICE // ZBUF):
            pltpu.make_async_copy(
                zbuf, chunk.at[pl.ds(s * SLICE + z * ZBUF, ZBUF)], zsem).wait()

        @pl.when(s == 0)
        def _():
            pltpu.make_async_copy(zbuf.at[pl.ds(0, BLK)],
                                  chunk.at[pl.ds(CHUNK, BLK)], zsem).wait()
        plsc.subcore_barrier()

        for w in range(NSUB):
            @pl.when(s == w)
            def _():
                def fire(b, _):
                    pltpu.async_copy(
                        big_f.at[pl.ds(b * BLK, BLK)],
                        chunk.at[big_i.at[pl.ds(b * BLK, BLK)]],
                        sem, add=True)
                    return 0
                lax.fori_loop(0, nblk, fire, 0)

                def drain(b, _):
                    pltpu.make_async_copy(
                        big_f.at[pl.ds(b * BLK, BLK)],
                        chunk.at[big_i.at[pl.ds(b * BLK, BLK)]],
                        sem).wait()
                    return 0
                lax.fori_loop(0, nblk, drain, 0)
            plsc.subcore_barrier()

        pltpu.async_copy(chunk.at[pl.ds(s * SLICE, SLICE)],
                         s_hbm.at[pl.ds(base + s * SLICE, SLICE)], osem)
        return 0

    lax.fori_loop(0, NPASS, chunk_pass, 0)

    last = (NPASS - 1) * NCORE + c
    pltpu.make_async_copy(chunk.at[pl.ds(s * SLICE, SLICE)],
                          s_hbm.at[pl.ds(last * CHUNK + s * SLICE, SLICE)],
                          osem).wait()


_sc_densify = functools.partial(
    pl.kernel,
    out_type=jax.ShapeDtypeStruct((SFLAT,), jnp.float32),
    mesh=plsc.VectorSubcoreMesh(core_axis_name="c", subcore_axis_name="s"),
    scratch_types=[
        pltpu.VMEM((BIGI,), jnp.int32),
        pltpu.VMEM((BIGF,), jnp.float32),
        pltpu.VMEM((ZBUF,), jnp.float32),
        pltpu.VMEM((48,), jnp.int32),
        pltpu.SMEM((2 * NPASS + 4,), jnp.int32),
        pltpu.VMEM_SHARED((NSUB * SPTILE,), jnp.int32),
        pltpu.VMEM_SHARED((NSUB * SPTILE,), jnp.float32),
        pltpu.VMEM_SHARED((CHUNK + BLK,), jnp.float32),
        pltpu.SemaphoreType.DMA,
        pltpu.SemaphoreType.DMA,
        pltpu.SemaphoreType.DMA,
    ],
)(_sc_densify_body)


BK = 128
NKB = IN_F // BK


def _mm_body(a_ref, s_ref, bias_ref, o_ref):
    kb = pl.program_id(0)

    @pl.when(kb == 0)
    def _():
        o_ref[...] = jnp.zeros_like(o_ref)

    a = a_ref[:, pl.ds(kb * BK, BK)].astype(jnp.bfloat16)
    b = s_ref[...].reshape(BK, OUT_F).astype(jnp.bfloat16)
    o_ref[...] += jnp.dot(a, b, preferred_element_type=jnp.float32)

    @pl.when(kb == NKB - 1)
    def _():
        acc = o_ref[...] + bias_ref[...]
        o_ref[...] = jnp.where(acc >= 0, acc, NEG_SLOPE * acc)


def _matmul(features, s_flat, bias2d):
    batch = features.shape[0]
    return pl.pallas_call(
        _mm_body,
        grid=(NKB,),
        in_specs=[
            pl.BlockSpec((batch, IN_F), lambda kb: (0, 0)),
            pl.BlockSpec((BK * OUT_F,), lambda kb: (kb,)),
            pl.BlockSpec((1, OUT_F), lambda kb: (0, 0)),
        ],
        out_specs=pl.BlockSpec((batch, OUT_F), lambda kb: (0, 0)),
        out_shape=jax.ShapeDtypeStruct((batch, OUT_F), jnp.float32),
    )(features, s_flat, bias2d)


def kernel(features, values, bias, rows, cols):
    nnz = rows.shape[0]
    pad = NNZ_PAD - nnz
    rows_p = jnp.concatenate([rows.astype(jnp.int32),
                              jnp.zeros((pad,), jnp.int32)])
    cols_p = jnp.concatenate([cols.astype(jnp.int32),
                              jnp.full((pad,), IN_F, jnp.int32)])
    vals_p = jnp.concatenate([values, jnp.zeros((pad,), jnp.float32)])
    s_flat = _sc_densify(rows_p, cols_p, vals_p)
    return _matmul(features, s_flat, bias.reshape(1, OUT_F))


# speedup vs baseline: 7.9718x; 1.0375x over previous
import functools

import jax
import jax.numpy as jnp
from jax import lax
from jax.experimental import pallas as pl
from jax.experimental.pallas import tpu as pltpu
from jax.experimental.pallas import tpu_sc as plsc

IN_F = 4096
OUT_F = 4096
NEG_SLOPE = 0.01

SFLAT = IN_F * OUT_F
NCHUNK = 16
CHUNK = SFLAT // NCHUNK
NPASS = NCHUNK // 2
NSUB = 16
NCORE = 2
SHARD_VECS = 656
SHARD = SHARD_VECS * 16
NNZ_PAD = SHARD * NSUB
SLICE = CHUNK // NSUB
ZBUF = 4096
BLK = 128
BINCAP = SHARD + NPASS * (BLK - 1) + 112
BINCAP = ((BINCAP + BLK - 1) // BLK) * BLK
SPTILE = BINCAP + BLK
WCAP = BINCAP + BLK
BIGI = max(2 * SHARD, WCAP)
BIGF = max(SHARD, WCAP)


def _sc_densify_body(rows_hbm, cols_hbm, vals_hbm, s_hbm,
                     big_i, big_f, zbuf, tmp, offs,
                     sp_bidx, sp_bval, chunk, sem, zsem, osem):
    c = lax.axis_index("c")
    s = lax.axis_index("s")
    shard0 = s * SHARD
    lane = lax.iota(jnp.int32, 16)
    rbase = s * SPTILE

    pltpu.async_copy(rows_hbm.at[pl.ds(shard0, SHARD)],
                     big_i.at[pl.ds(0, SHARD)], sem)
    pltpu.async_copy(cols_hbm.at[pl.ds(shard0, SHARD)],
                     big_i.at[pl.ds(SHARD, SHARD)], sem)
    pltpu.async_copy(vals_hbm.at[pl.ds(shard0, SHARD)],
                     big_f.at[pl.ds(0, SHARD)], sem)

    def zb(i, _):
        zbuf[pl.ds(i * 16, 16)] = jnp.zeros((16,), jnp.float32)
        return 0
    lax.fori_loop(0, ZBUF // 16, zb, 0)

    pltpu.make_async_copy(rows_hbm.at[pl.ds(shard0, SHARD)],
                          big_i.at[pl.ds(0, SHARD)], sem).wait()
    pltpu.make_async_copy(cols_hbm.at[pl.ds(shard0, SHARD)],
                          big_i.at[pl.ds(SHARD, SHARD)], sem).wait()
    pltpu.make_async_copy(vals_hbm.at[pl.ds(shard0, SHARD)],
                          big_f.at[pl.ds(0, SHARD)], sem).wait()

    def ha(i, cnts):
        r = big_i[pl.ds(i * 16, 16)]
        cc = big_i[pl.ds(SHARD + i * 16, 16)]
        ch = (cc * OUT_F + r) >> 20
        one = jnp.ones((16,), jnp.int32)
        zero = jnp.zeros((16,), jnp.int32)
        return tuple(cnts[q] + jnp.where(ch == (q * NCORE + c), one, zero)
                     for q in range(NPASS))
    cnts = lax.fori_loop(0, SHARD_VECS, ha,
                         tuple(jnp.zeros((16,), jnp.int32)
                               for _ in range(NPASS)))

    starts = []
    bq = jnp.int32(0)
    for q in range(NPASS):
        x = cnts[q]
        for sh in (1, 2, 4, 8):
            tmp[pl.ds(16, 16)] = x
            tmp[pl.ds(0, 16)] = jnp.zeros((16,), jnp.int32)
            x = x + tmp[pl.ds(16 - sh, 16)]
        tmp[pl.ds(16, 16)] = x
        tmp[pl.ds(0, 16)] = jnp.zeros((16,), jnp.int32)
        excl = tmp[pl.ds(15, 16)]
        tot = x[15]
        offs[q] = bq
        offs[NPASS + 1 + q] = tot
        starts.append(excl + (rbase + bq * BLK))
        bq = bq + ((tot + BLK - 1) >> 7)
    offs[NPASS] = bq

    dump = rbase + BINCAP + lane

    def hb(i, sts):
        r = big_i[pl.ds(i * 16, 16)]
        cc = big_i[pl.ds(SHARD + i * 16, 16)]
        flat = cc * OUT_F + r
        ch = flat >> 20
        rel = flat & (CHUNK - 1)
        one = jnp.ones((16,), jnp.int32)
        zero = jnp.zeros((16,), jnp.int32)
        pos = dump
        out = []
        for q in range(NPASS):
            m = ch == (q * NCORE + c)
            pos = jnp.where(m, sts[q], pos)
            out.append(sts[q] + jnp.where(m, one, zero))
        big_i[pl.ds(i * 16, 16)] = rel
        big_i[pl.ds(SHARD + i * 16, 16)] = pos
        return tuple(out)
    lax.fori_loop(0, SHARD_VECS, hb, tuple(starts))

    def pfire(b, _):
        idx = big_i.at[pl.ds(SHARD + b * BLK, BLK)]
        pltpu.async_copy(big_i.at[pl.ds(b * BLK, BLK)], sp_bidx.at[idx], sem)
        pltpu.async_copy(big_f.at[pl.ds(b * BLK, BLK)], sp_bval.at[idx], sem)
        return 0
    lax.fori_loop(0, SHARD // BLK, pfire, 0)

    def pdrain(b, _):
        idx = big_i.at[pl.ds(SHARD + b * BLK, BLK)]
        pltpu.make_async_copy(big_i.at[pl.ds(b * BLK, BLK)],
                              sp_bidx.at[idx], sem).wait()
        pltpu.make_async_copy(big_f.at[pl.ds(b * BLK, BLK)],
                              sp_bval.at[idx], sem).wait()
        return 0
    lax.fori_loop(0, SHARD // BLK, pdrain, 0)

    def _settle(i, acc):
        return acc + i * i
    offs[2 * NPASS + 1] = lax.fori_loop(0, 1000, _settle, jnp.int32(0))
    plsc.subcore_barrier()

    def chunk_pass(q, _):
        chunk_id = q * NCORE + c
        base = chunk_id * CHUNK

        @pl.when(q > 0)
        def _():
            prev = (q - 1) * NCORE + c
            pltpu.make_async_copy(
                chunk.at[pl.ds(s * SLICE, SLICE)],
                s_hbm.at[pl.ds(prev * CHUNK + s * SLICE, SLICE)], osem).wait()

        for z in range(SLICE // ZBUF):
            pltpu.async_copy(zbuf, chunk.at[pl.ds(s * SLICE + z * ZBUF, ZBUF)],
                             zsem)
        @pl.when(s == 0)
        def _():
            pltpu.async_copy(zbuf.at[pl.ds(0, BLK)],
                             chunk.at[pl.ds(CHUNK, BLK)], zsem)

        b_lo = offs[q]
        tot = offs[NPASS + 1 + q]
        nblk = (tot + BLK - 1) >> 7

        def cin(b, _):
            pltpu.async_copy(
                sp_bidx.at[pl.ds(rbase + (b_lo + b) * BLK, BLK)],
                big_i.at[pl.ds(b * BLK, BLK)], sem)
            pltpu.async_copy(
                sp_bval.at[pl.ds(rbase + (b_lo + b) * BLK, BLK)],
                big_f.at[pl.ds(b * BLK, BLK)], sem)
            return 0
        lax.fori_loop(0, nblk, cin, 0)

        def cin_drain(b, _):
            pltpu.make_async_copy(
                sp_bidx.at[pl.ds(rbase + (b_lo + b) * BLK, BLK)],
                big_i.at[pl.ds(b * BLK, BLK)], sem).wait()
            pltpu.make_async_copy(
                sp_bval.at[pl.ds(rbase + (b_lo + b) * BLK, BLK)],
                big_f.at[pl.ds(b * BLK, BLK)], sem).wait()
            return 0
        lax.fori_loop(0, nblk, cin_drain, 0)

        def tfill(j, _):
            p = tot + j * 16
            big_i[pl.ds(p, 16)] = CHUNK + ((p + lane) & (BLK - 1))
            return 0
        lax.fori_loop(0, BLK // 16, tfill, 0)

        for z in range(SLICE // ZBUF):
            pltpu.make_async_copy(
                zbuf, chunk.at[pl.ds(s * SLICE + z * ZBUF, ZBUF)], zsem).wait()

        @pl.when(s == 0)
        def _():
            pltpu.make_async_copy(zbuf.at[pl.ds(0, BLK)],
                                  chunk.at[pl.ds(CHUNK, BLK)], zsem).wait()
        plsc.subcore_barrier()

        for w in range(NSUB):
            @pl.when(s == w)
            def _():
                def fire(b, _):
                    pltpu.async_copy(
                        big_f.at[pl.ds(b * BLK, BLK)],
                        chunk.at[big_i.at[pl.ds(b * BLK, BLK)]],
                        sem, add=True)
                    return 0
                lax.fori_loop(0, nblk, fire, 0)

                def drain(b, _):
                    pltpu.make_async_copy(
                        big_f.at[pl.ds(b * BLK, BLK)],
                        chunk.at[big_i.at[pl.ds(b * BLK, BLK)]],
                        sem).wait()
                    return 0
                lax.fori_loop(0, nblk, drain, 0)
            plsc.subcore_barrier()

        pltpu.async_copy(chunk.at[pl.ds(s * SLICE, SLICE)],
                         s_hbm.at[pl.ds(base + s * SLICE, SLICE)], osem)
        return 0

    lax.fori_loop(0, NPASS, chunk_pass, 0)

    last = (NPASS - 1) * NCORE + c
    pltpu.make_async_copy(chunk.at[pl.ds(s * SLICE, SLICE)],
                          s_hbm.at[pl.ds(last * CHUNK + s * SLICE, SLICE)],
                          osem).wait()


_sc_densify = functools.partial(
    pl.kernel,
    out_type=jax.ShapeDtypeStruct((SFLAT,), jnp.float32),
    mesh=plsc.VectorSubcoreMesh(core_axis_name="c", subcore_axis_name="s"),
    scratch_types=[
        pltpu.VMEM((BIGI,), jnp.int32),
        pltpu.VMEM((BIGF,), jnp.float32),
        pltpu.VMEM((ZBUF,), jnp.float32),
        pltpu.VMEM((48,), jnp.int32),
        pltpu.SMEM((2 * NPASS + 4,), jnp.int32),
        pltpu.VMEM_SHARED((NSUB * SPTILE,), jnp.int32),
        pltpu.VMEM_SHARED((NSUB * SPTILE,), jnp.float32),
        pltpu.VMEM_SHARED((CHUNK + BLK,), jnp.float32),
        pltpu.SemaphoreType.DMA,
        pltpu.SemaphoreType.DMA,
        pltpu.SemaphoreType.DMA,
    ],
)(_sc_densify_body)


BK = 256
NKB = IN_F // BK


def _mm_body(a_ref, s_ref, bias_ref, o_ref):
    kb = pl.program_id(0)

    @pl.when(kb == 0)
    def _():
        o_ref[...] = jnp.zeros_like(o_ref)

    a = a_ref[:, pl.ds(kb * BK, BK)].astype(jnp.bfloat16)
    b = s_ref[...].reshape(BK, OUT_F).astype(jnp.bfloat16)
    o_ref[...] += jnp.dot(a, b, preferred_element_type=jnp.float32)

    @pl.when(kb == NKB - 1)
    def _():
        acc = o_ref[...] + bias_ref[...]
        o_ref[...] = jnp.where(acc >= 0, acc, NEG_SLOPE * acc)


def _matmul(features, s_flat, bias2d):
    batch = features.shape[0]
    return pl.pallas_call(
        _mm_body,
        grid=(NKB,),
        in_specs=[
            pl.BlockSpec((batch, IN_F), lambda kb: (0, 0)),
            pl.BlockSpec((BK * OUT_F,), lambda kb: (kb,)),
            pl.BlockSpec((1, OUT_F), lambda kb: (0, 0)),
        ],
        out_specs=pl.BlockSpec((batch, OUT_F), lambda kb: (0, 0)),
        out_shape=jax.ShapeDtypeStruct((batch, OUT_F), jnp.float32),
    )(features, s_flat, bias2d)


def kernel(features, values, bias, rows, cols):
    nnz = rows.shape[0]
    pad = NNZ_PAD - nnz
    rows_p = jnp.concatenate([rows.astype(jnp.int32),
                              jnp.zeros((pad,), jnp.int32)])
    cols_p = jnp.concatenate([cols.astype(jnp.int32),
                              jnp.full((pad,), IN_F, jnp.int32)])
    vals_p = jnp.concatenate([values, jnp.zeros((pad,), jnp.float32)])
    s_flat = _sc_densify(rows_p, cols_p, vals_p)
    return _matmul(features, s_flat, bias.reshape(1, OUT_F))


# speedup vs baseline: 8.1343x; 1.0204x over previous
import functools

import jax
import jax.numpy as jnp
from jax import lax
from jax.experimental import pallas as pl
from jax.experimental.pallas import tpu as pltpu
from jax.experimental.pallas import tpu_sc as plsc

IN_F = 4096
OUT_F = 4096
NEG_SLOPE = 0.01

SFLAT = IN_F * OUT_F
NCHUNK = 16
CHUNK = SFLAT // NCHUNK
NPASS = NCHUNK // 2
NSUB = 16
NCORE = 2
SHARD_VECS = 656
SHARD = SHARD_VECS * 16
NNZ_PAD = SHARD * NSUB
SLICE = CHUNK // NSUB
ZBUF = 4096
BLK = 128
BINCAP = SHARD + NPASS * (BLK - 1) + 112
BINCAP = ((BINCAP + BLK - 1) // BLK) * BLK
SPTILE = BINCAP + BLK
WCAP = BINCAP + BLK
BIGI = max(2 * SHARD, WCAP)
BIGF = max(SHARD, WCAP)


def _sc_densify_body(rows_hbm, cols_hbm, vals_hbm, s_hbm,
                     big_i, big_f, zbuf, tmp, offs,
                     sp_bidx, sp_bval, chunk, sem, zsem, osem):
    c = lax.axis_index("c")
    s = lax.axis_index("s")
    shard0 = s * SHARD
    lane = lax.iota(jnp.int32, 16)
    rbase = s * SPTILE

    pltpu.async_copy(rows_hbm.at[pl.ds(shard0, SHARD)],
                     big_i.at[pl.ds(0, SHARD)], sem)
    pltpu.async_copy(cols_hbm.at[pl.ds(shard0, SHARD)],
                     big_i.at[pl.ds(SHARD, SHARD)], sem)
    pltpu.async_copy(vals_hbm.at[pl.ds(shard0, SHARD)],
                     big_f.at[pl.ds(0, SHARD)], sem)

    def zb(i, _):
        zbuf[pl.ds(i * 16, 16)] = jnp.zeros((16,), jnp.float32)
        return 0
    lax.fori_loop(0, ZBUF // 16, zb, 0)

    pltpu.make_async_copy(rows_hbm.at[pl.ds(shard0, SHARD)],
                          big_i.at[pl.ds(0, SHARD)], sem).wait()
    pltpu.make_async_copy(cols_hbm.at[pl.ds(shard0, SHARD)],
                          big_i.at[pl.ds(SHARD, SHARD)], sem).wait()
    pltpu.make_async_copy(vals_hbm.at[pl.ds(shard0, SHARD)],
                          big_f.at[pl.ds(0, SHARD)], sem).wait()

    def ha(i, cnts):
        r = big_i[pl.ds(i * 16, 16)]
        cc = big_i[pl.ds(SHARD + i * 16, 16)]
        ch = (cc * OUT_F + r) >> 20
        one = jnp.ones((16,), jnp.int32)
        zero = jnp.zeros((16,), jnp.int32)
        return tuple(cnts[q] + jnp.where(ch == (q * NCORE + c), one, zero)
                     for q in range(NPASS))
    cnts = lax.fori_loop(0, SHARD_VECS, ha,
                         tuple(jnp.zeros((16,), jnp.int32)
                               for _ in range(NPASS)))

    starts = []
    bq = jnp.int32(0)
    for q in range(NPASS):
        x = cnts[q]
        for sh in (1, 2, 4, 8):
            tmp[pl.ds(16, 16)] = x
            tmp[pl.ds(0, 16)] = jnp.zeros((16,), jnp.int32)
            x = x + tmp[pl.ds(16 - sh, 16)]
        tmp[pl.ds(16, 16)] = x
        tmp[pl.ds(0, 16)] = jnp.zeros((16,), jnp.int32)
        excl = tmp[pl.ds(15, 16)]
        tot = x[15]
        offs[q] = bq
        offs[NPASS + 1 + q] = tot
        starts.append(excl + (rbase + bq * BLK))
        bq = bq + ((tot + BLK - 1) >> 7)
    offs[NPASS] = bq

    dump = rbase + BINCAP + lane

    def hb(i, sts):
        r = big_i[pl.ds(i * 16, 16)]
        cc = big_i[pl.ds(SHARD + i * 16, 16)]
        flat = cc * OUT_F + r
        ch = flat >> 20
        rel = flat & (CHUNK - 1)
        one = jnp.ones((16,), jnp.int32)
        zero = jnp.zeros((16,), jnp.int32)
        pos = dump
        out = []
        for q in range(NPASS):
            m = ch == (q * NCORE + c)
            pos = jnp.where(m, sts[q], pos)
            out.append(sts[q] + jnp.where(m, one, zero))
        big_i[pl.ds(i * 16, 16)] = rel
        big_i[pl.ds(SHARD + i * 16, 16)] = pos
        return tuple(out)
    lax.fori_loop(0, SHARD_VECS, hb, tuple(starts))

    def pfire(b, _):
        idx = big_i.at[pl.ds(SHARD + b * BLK, BLK)]
        pltpu.async_copy(big_i.at[pl.ds(b * BLK, BLK)], sp_bidx.at[idx], sem)
        pltpu.async_copy(big_f.at[pl.ds(b * BLK, BLK)], sp_bval.at[idx], sem)
        return 0
    lax.fori_loop(0, SHARD // BLK, pfire, 0)

    def pdrain(b, _):
        idx = big_i.at[pl.ds(SHARD + b * BLK, BLK)]
        pltpu.make_async_copy(big_i.at[pl.ds(b * BLK, BLK)],
                              sp_bidx.at[idx], sem).wait()
        pltpu.make_async_copy(big_f.at[pl.ds(b * BLK, BLK)],
                              sp_bval.at[idx], sem).wait()
        return 0
    lax.fori_loop(0, SHARD // BLK, pdrain, 0)

    def _settle(i, acc):
        return acc + i * i
    offs[2 * NPASS + 1] = lax.fori_loop(0, 1000, _settle, jnp.int32(0))
    plsc.subcore_barrier()

    def chunk_pass(q, _):
        chunk_id = q * NCORE + c
        base = chunk_id * CHUNK

        @pl.when(q > 0)
        def _():
            prev = (q - 1) * NCORE + c
            pltpu.make_async_copy(
                chunk.at[pl.ds(s * SLICE, SLICE)],
                s_hbm.at[pl.ds(prev * CHUNK + s * SLICE, SLICE)], osem).wait()

        for z in range(SLICE // ZBUF):
            pltpu.async_copy(zbuf, chunk.at[pl.ds(s * SLICE + z * ZBUF, ZBUF)],
                             zsem)
        @pl.when(s == 0)
        def _():
            pltpu.async_copy(zbuf.at[pl.ds(0, BLK)],
                             chunk.at[pl.ds(CHUNK, BLK)], zsem)

        b_lo = offs[q]
        tot = offs[NPASS + 1 + q]
        nblk = (tot + BLK - 1) >> 7

        def cin(b, _):
            pltpu.async_copy(
                sp_bidx.at[pl.ds(rbase + (b_lo + b) * BLK, BLK)],
                big_i.at[pl.ds(b * BLK, BLK)], sem)
            pltpu.async_copy(
                sp_bval.at[pl.ds(rbase + (b_lo + b) * BLK, BLK)],
                big_f.at[pl.ds(b * BLK, BLK)], sem)
            return 0
        lax.fori_loop(0, nblk, cin, 0)

        def cin_drain(b, _):
            pltpu.make_async_copy(
                sp_bidx.at[pl.ds(rbase + (b_lo + b) * BLK, BLK)],
                big_i.at[pl.ds(b * BLK, BLK)], sem).wait()
            pltpu.make_async_copy(
                sp_bval.at[pl.ds(rbase + (b_lo + b) * BLK, BLK)],
                big_f.at[pl.ds(b * BLK, BLK)], sem).wait()
            return 0
        lax.fori_loop(0, nblk, cin_drain, 0)

        def tfill(j, _):
            p = tot + j * 16
            big_i[pl.ds(p, 16)] = CHUNK + ((p + lane) & (BLK - 1))
            return 0
        lax.fori_loop(0, BLK // 16, tfill, 0)

        for z in range(SLICE // ZBUF):
            pltpu.make_async_copy(
                zbuf, chunk.at[pl.ds(s * SLICE + z * ZBUF, ZBUF)], zsem).wait()

        @pl.when(s == 0)
        def _():
            pltpu.make_async_copy(zbuf.at[pl.ds(0, BLK)],
                                  chunk.at[pl.ds(CHUNK, BLK)], zsem).wait()
        plsc.subcore_barrier()

        for w in range(NSUB):
            @pl.when(s == w)
            def _():
                def fire(b, _):
                    pltpu.async_copy(
                        big_f.at[pl.ds(b * BLK, BLK)],
                        chunk.at[big_i.at[pl.ds(b * BLK, BLK)]],
                        sem, add=True)
                    return 0
                lax.fori_loop(0, nblk, fire, 0)

                def drain(b, _):
                    pltpu.make_async_copy(
                        big_f.at[pl.ds(b * BLK, BLK)],
                        chunk.at[big_i.at[pl.ds(b * BLK, BLK)]],
                        sem).wait()
                    return 0
                lax.fori_loop(0, nblk, drain, 0)
            plsc.subcore_barrier()

        pltpu.async_copy(chunk.at[pl.ds(s * SLICE, SLICE)],
                         s_hbm.at[pl.ds(base + s * SLICE, SLICE)], osem)
        return 0

    lax.fori_loop(0, NPASS, chunk_pass, 0)

    last = (NPASS - 1) * NCORE + c
    pltpu.make_async_copy(chunk.at[pl.ds(s * SLICE, SLICE)],
                          s_hbm.at[pl.ds(last * CHUNK + s * SLICE, SLICE)],
                          osem).wait()


_sc_densify = functools.partial(
    pl.kernel,
    out_type=jax.ShapeDtypeStruct((SFLAT,), jnp.float32),
    mesh=plsc.VectorSubcoreMesh(core_axis_name="c", subcore_axis_name="s"),
    scratch_types=[
        pltpu.VMEM((BIGI,), jnp.int32),
        pltpu.VMEM((BIGF,), jnp.float32),
        pltpu.VMEM((ZBUF,), jnp.float32),
        pltpu.VMEM((48,), jnp.int32),
        pltpu.SMEM((2 * NPASS + 4,), jnp.int32),
        pltpu.VMEM_SHARED((NSUB * SPTILE,), jnp.int32),
        pltpu.VMEM_SHARED((NSUB * SPTILE,), jnp.float32),
        pltpu.VMEM_SHARED((CHUNK + BLK,), jnp.float32),
        pltpu.SemaphoreType.DMA,
        pltpu.SemaphoreType.DMA,
        pltpu.SemaphoreType.DMA,
    ],
)(_sc_densify_body)


BK = 512
NKB = IN_F // BK


def _mm_body(a_ref, s_ref, bias_ref, o_ref):
    kb = pl.program_id(0)

    @pl.when(kb == 0)
    def _():
        o_ref[...] = jnp.zeros_like(o_ref)

    a = a_ref[:, pl.ds(kb * BK, BK)].astype(jnp.bfloat16)
    b = s_ref[...].reshape(BK, OUT_F).astype(jnp.bfloat16)
    o_ref[...] += jnp.dot(a, b, preferred_element_type=jnp.float32)

    @pl.when(kb == NKB - 1)
    def _():
        acc = o_ref[...] + bias_ref[...]
        o_ref[...] = jnp.where(acc >= 0, acc, NEG_SLOPE * acc)


def _matmul(features, s_flat, bias2d):
    batch = features.shape[0]
    return pl.pallas_call(
        _mm_body,
        grid=(NKB,),
        in_specs=[
            pl.BlockSpec((batch, IN_F), lambda kb: (0, 0)),
            pl.BlockSpec((BK * OUT_F,), lambda kb: (kb,)),
            pl.BlockSpec((1, OUT_F), lambda kb: (0, 0)),
        ],
        out_specs=pl.BlockSpec((batch, OUT_F), lambda kb: (0, 0)),
        out_shape=jax.ShapeDtypeStruct((batch, OUT_F), jnp.float32),
    )(features, s_flat, bias2d)


def kernel(features, values, bias, rows, cols):
    nnz = rows.shape[0]
    pad = NNZ_PAD - nnz
    rows_p = jnp.concatenate([rows.astype(jnp.int32),
                              jnp.zeros((pad,), jnp.int32)])
    cols_p = jnp.concatenate([cols.astype(jnp.int32),
                              jnp.full((pad,), IN_F, jnp.int32)])
    vals_p = jnp.concatenate([values, jnp.zeros((pad,), jnp.float32)])
    s_flat = _sc_densify(rows_p, cols_p, vals_p)
    return _matmul(features, s_flat, bias.reshape(1, OUT_F))


# speedup vs baseline: 8.1531x; 1.0023x over previous
import functools

import jax
import jax.numpy as jnp
from jax import lax
from jax.experimental import pallas as pl
from jax.experimental.pallas import tpu as pltpu
from jax.experimental.pallas import tpu_sc as plsc

IN_F = 4096
OUT_F = 4096
NEG_SLOPE = 0.01

SFLAT = IN_F * OUT_F
NCHUNK = 16
CHUNK = SFLAT // NCHUNK
NPASS = NCHUNK // 2
NSUB = 16
NCORE = 2
SHARD_VECS = 656
SHARD = SHARD_VECS * 16
NNZ_PAD = SHARD * NSUB
SLICE = CHUNK // NSUB
ZBUF = 4096
BLK = 128
BINCAP = SHARD + NPASS * (BLK - 1) + 112
BINCAP = ((BINCAP + BLK - 1) // BLK) * BLK
SPTILE = BINCAP + BLK
WCAP = BINCAP + BLK
BIGI = max(2 * SHARD, WCAP)
BIGF = max(SHARD, WCAP)


def _sc_densify_body(rows_hbm, cols_hbm, vals_hbm, s_hbm,
                     big_i, big_f, zbuf, tmp, offs,
                     sp_bidx, sp_bval, chunk, sem, zsem, osem):
    c = lax.axis_index("c")
    s = lax.axis_index("s")
    shard0 = s * SHARD
    lane = lax.iota(jnp.int32, 16)
    rbase = s * SPTILE

    pltpu.async_copy(rows_hbm.at[pl.ds(shard0, SHARD)],
                     big_i.at[pl.ds(0, SHARD)], sem)
    pltpu.async_copy(cols_hbm.at[pl.ds(shard0, SHARD)],
                     big_i.at[pl.ds(SHARD, SHARD)], sem)
    pltpu.async_copy(vals_hbm.at[pl.ds(shard0, SHARD)],
                     big_f.at[pl.ds(0, SHARD)], sem)

    def zb(i, _):
        zbuf[pl.ds(i * 16, 16)] = jnp.zeros((16,), jnp.float32)
        return 0
    lax.fori_loop(0, ZBUF // 16, zb, 0)

    pltpu.make_async_copy(rows_hbm.at[pl.ds(shard0, SHARD)],
                          big_i.at[pl.ds(0, SHARD)], sem).wait()
    pltpu.make_async_copy(cols_hbm.at[pl.ds(shard0, SHARD)],
                          big_i.at[pl.ds(SHARD, SHARD)], sem).wait()
    pltpu.make_async_copy(vals_hbm.at[pl.ds(shard0, SHARD)],
                          big_f.at[pl.ds(0, SHARD)], sem).wait()

    def ha(i, cnts):
        r = big_i[pl.ds(i * 16, 16)]
        cc = big_i[pl.ds(SHARD + i * 16, 16)]
        ch = (cc * OUT_F + r) >> 20
        one = jnp.ones((16,), jnp.int32)
        zero = jnp.zeros((16,), jnp.int32)
        return tuple(cnts[q] + jnp.where(ch == (q * NCORE + c), one, zero)
                     for q in range(NPASS))
    cnts = lax.fori_loop(0, SHARD_VECS, ha,
                         tuple(jnp.zeros((16,), jnp.int32)
                               for _ in range(NPASS)))

    starts = []
    bq = jnp.int32(0)
    for q in range(NPASS):
        x = cnts[q]
        for sh in (1, 2, 4, 8):
            tmp[pl.ds(16, 16)] = x
            tmp[pl.ds(0, 16)] = jnp.zeros((16,), jnp.int32)
            x = x + tmp[pl.ds(16 - sh, 16)]
        tmp[pl.ds(16, 16)] = x
        tmp[pl.ds(0, 16)] = jnp.zeros((16,), jnp.int32)
        excl = tmp[pl.ds(15, 16)]
        tot = x[15]
        offs[q] = bq
        offs[NPASS + 1 + q] = tot
        starts.append(excl + (rbase + bq * BLK))
        bq = bq + ((tot + BLK - 1) >> 7)
    offs[NPASS] = bq

    dump = rbase + BINCAP + lane

    def hb(i, sts):
        r = big_i[pl.ds(i * 16, 16)]
        cc = big_i[pl.ds(SHARD + i * 16, 16)]
        flat = cc * OUT_F + r
        ch = flat >> 20
        rel = flat & (CHUNK - 1)
        one = jnp.ones((16,), jnp.int32)
        zero = jnp.zeros((16,), jnp.int32)
        pos = dump
        out = []
        for q in range(NPASS):
            m = ch == (q * NCORE + c)
            pos = jnp.where(m, sts[q], pos)
            out.append(sts[q] + jnp.where(m, one, zero))
        big_i[pl.ds(i * 16, 16)] = rel
        big_i[pl.ds(SHARD + i * 16, 16)] = pos
        return tuple(out)
    lax.fori_loop(0, SHARD_VECS, hb, tuple(starts))

    def pfire(b, _):
        idx = big_i.at[pl.ds(SHARD + b * BLK, BLK)]
        pltpu.async_copy(big_i.at[pl.ds(b * BLK, BLK)], sp_bidx.at[idx], sem)
        pltpu.async_copy(big_f.at[pl.ds(b * BLK, BLK)], sp_bval.at[idx], sem)
        return 0
    lax.fori_loop(0, SHARD // BLK, pfire, 0)

    def pdrain(b, _):
        idx = big_i.at[pl.ds(SHARD + b * BLK, BLK)]
        pltpu.make_async_copy(big_i.at[pl.ds(b * BLK, BLK)],
                              sp_bidx.at[idx], sem).wait()
        pltpu.make_async_copy(big_f.at[pl.ds(b * BLK, BLK)],
                              sp_bval.at[idx], sem).wait()
        return 0
    lax.fori_loop(0, SHARD // BLK, pdrain, 0)

    def _settle(i, acc):
        return acc + i * i
    offs[2 * NPASS + 1] = lax.fori_loop(0, 1000, _settle, jnp.int32(0))
    plsc.subcore_barrier()

    def chunk_pass(q, _):
        chunk_id = q * NCORE + c
        base = chunk_id * CHUNK

        @pl.when(q > 0)
        def _():
            prev = (q - 1) * NCORE + c
            pltpu.make_async_copy(
                chunk.at[pl.ds(s * SLICE, SLICE)],
                s_hbm.at[pl.ds(prev * CHUNK + s * SLICE, SLICE)], osem).wait()

        for z in range(SLICE // ZBUF):
            pltpu.async_copy(zbuf, chunk.at[pl.ds(s * SLICE + z * ZBUF, ZBUF)],
                             zsem)
        @pl.when(s == 0)
        def _():
            pltpu.async_copy(zbuf.at[pl.ds(0, BLK)],
                             chunk.at[pl.ds(CHUNK, BLK)], zsem)

        b_lo = offs[q]
        tot = offs[NPASS + 1 + q]
        nblk = (tot + BLK - 1) >> 7

        def cin(b, _):
            pltpu.async_copy(
                sp_bidx.at[pl.ds(rbase + (b_lo + b) * BLK, BLK)],
                big_i.at[pl.ds(b * BLK, BLK)], sem)
            pltpu.async_copy(
                sp_bval.at[pl.ds(rbase + (b_lo + b) * BLK, BLK)],
                big_f.at[pl.ds(b * BLK, BLK)], sem)
            return 0
        lax.fori_loop(0, nblk, cin, 0)

        def cin_drain(b, _):
            pltpu.make_async_copy(
                sp_bidx.at[pl.ds(rbase + (b_lo + b) * BLK, BLK)],
                big_i.at[pl.ds(b * BLK, BLK)], sem).wait()
            pltpu.make_async_copy(
                sp_bval.at[pl.ds(rbase + (b_lo + b) * BLK, BLK)],
                big_f.at[pl.ds(b * BLK, BLK)], sem).wait()
            return 0
        lax.fori_loop(0, nblk, cin_drain, 0)

        def tfill(j, _):
            p = tot + j * 16
            big_i[pl.ds(p, 16)] = CHUNK + ((p + lane) & (BLK - 1))
            return 0
        lax.fori_loop(0, BLK // 16, tfill, 0)

        for z in range(SLICE // ZBUF):
            pltpu.make_async_copy(
                zbuf, chunk.at[pl.ds(s * SLICE + z * ZBUF, ZBUF)], zsem).wait()

        @pl.when(s == 0)
        def _():
            pltpu.make_async_copy(zbuf.at[pl.ds(0, BLK)],
                                  chunk.at[pl.ds(CHUNK, BLK)], zsem).wait()
        plsc.subcore_barrier()

        for w in range(NSUB):
            @pl.when(s == w)
            def _():
                def fire(b, _):
                    pltpu.async_copy(
                        big_f.at[pl.ds(b * BLK, BLK)],
                        chunk.at[big_i.at[pl.ds(b * BLK, BLK)]],
                        sem, add=True)
                    return 0
                lax.fori_loop(0, nblk, fire, 0)

                def drain(b, _):
                    pltpu.make_async_copy(
                        big_f.at[pl.ds(b * BLK, BLK)],
                        chunk.at[big_i.at[pl.ds(b * BLK, BLK)]],
                        sem).wait()
                    return 0
                lax.fori_loop(0, nblk, drain, 0)
            plsc.subcore_barrier()

        pltpu.async_copy(chunk.at[pl.ds(s * SLICE, SLICE)],
                         s_hbm.at[pl.ds(base + s * SLICE, SLICE)], osem)
        return 0

    lax.fori_loop(0, NPASS, chunk_pass, 0)

    last = (NPASS - 1) * NCORE + c
    pltpu.make_async_copy(chunk.at[pl.ds(s * SLICE, SLICE)],
                          s_hbm.at[pl.ds(last * CHUNK + s * SLICE, SLICE)],
                          osem).wait()


_sc_densify = functools.partial(
    pl.kernel,
    out_type=jax.ShapeDtypeStruct((SFLAT,), jnp.float32),
    mesh=plsc.VectorSubcoreMesh(core_axis_name="c", subcore_axis_name="s"),
    scratch_types=[
        pltpu.VMEM((BIGI,), jnp.int32),
        pltpu.VMEM((BIGF,), jnp.float32),
        pltpu.VMEM((ZBUF,), jnp.float32),
        pltpu.VMEM((48,), jnp.int32),
        pltpu.SMEM((2 * NPASS + 4,), jnp.int32),
        pltpu.VMEM_SHARED((NSUB * SPTILE,), jnp.int32),
        pltpu.VMEM_SHARED((NSUB * SPTILE,), jnp.float32),
        pltpu.VMEM_SHARED((CHUNK + BLK,), jnp.float32),
        pltpu.SemaphoreType.DMA,
        pltpu.SemaphoreType.DMA,
        pltpu.SemaphoreType.DMA,
    ],
)(_sc_densify_body)


BK = 1024
NKB = IN_F // BK


def _mm_body(a_ref, s_ref, bias_ref, o_ref):
    kb = pl.program_id(0)

    @pl.when(kb == 0)
    def _():
        o_ref[...] = jnp.zeros_like(o_ref)

    a = a_ref[:, pl.ds(kb * BK, BK)].astype(jnp.bfloat16)
    b = s_ref[...].reshape(BK, OUT_F).astype(jnp.bfloat16)
    o_ref[...] += jnp.dot(a, b, preferred_element_type=jnp.float32)

    @pl.when(kb == NKB - 1)
    def _():
        acc = o_ref[...] + bias_ref[...]
        o_ref[...] = jnp.where(acc >= 0, acc, NEG_SLOPE * acc)


def _matmul(features, s_flat, bias2d):
    batch = features.shape[0]
    return pl.pallas_call(
        _mm_body,
        grid=(NKB,),
        in_specs=[
            pl.BlockSpec((batch, IN_F), lambda kb: (0, 0)),
            pl.BlockSpec((BK * OUT_F,), lambda kb: (kb,)),
            pl.BlockSpec((1, OUT_F), lambda kb: (0, 0)),
        ],
        out_specs=pl.BlockSpec((batch, OUT_F), lambda kb: (0, 0)),
        out_shape=jax.ShapeDtypeStruct((batch, OUT_F), jnp.float32),
    )(features, s_flat, bias2d)


def kernel(features, values, bias, rows, cols):
    nnz = rows.shape[0]
    pad = NNZ_PAD - nnz
    rows_p = jnp.concatenate([rows.astype(jnp.int32),
                              jnp.zeros((pad,), jnp.int32)])
    cols_p = jnp.concatenate([cols.astype(jnp.int32),
                              jnp.full((pad,), IN_F, jnp.int32)])
    vals_p = jnp.concatenate([values, jnp.zeros((pad,), jnp.float32)])
    s_flat = _sc_densify(rows_p, cols_p, vals_p)
    return _matmul(features, s_flat, bias.reshape(1, OUT_F))
